# Initial kernel scaffold; baseline (speedup 1.0000x reference)
import jax
import jax.numpy as jnp
from jax.experimental import pallas as pl


def kernel(x, edge_index, edge_attr, w_node, b_node, we0, be0, w10, b10, w20, b20, eps0, we1, be1, w11, b11, w21, b21, eps1, w_out, b_out):
    raise NotImplementedError("write your pallas kernel here")


# speedup vs baseline: 2.7606x; 2.7606x over previous
import functools

import jax
import jax.numpy as jnp
from jax import lax
from jax.experimental import pallas as pl
from jax.experimental.pallas import tpu as pltpu
from jax.experimental.pallas import tpu_sc as plsc

_N = 10000
_E = 320000
_H = 128
_NC = 2
_NS = 16
_NW = _NC * _NS
_EPW = _E // _NW
_K = 80
_NCH = _EPW // _K
_NP = 10240
_RPT = _NP // _NS
_ZR = 32
_NGRP = 5
_GCH = _NCH // _NGRP
_GE = _GCH * _K
_BN = 1000
_BE = 2000


def _node_enc_body(x_ref, w_ref, b_ref, o_ref):
  o_ref[...] = (
      jnp.dot(x_ref[...], w_ref[...], preferred_element_type=jnp.float32)
      + b_ref[...]
  )


def _node_encode(x, w, b):
  d_in = x.shape[1]
  return pl.pallas_call(
      _node_enc_body,
      grid=(_N // _BN,),
      in_specs=[
          pl.BlockSpec((_BN, d_in), lambda i: (i, 0)),
          pl.BlockSpec((d_in, _H), lambda i: (0, 0)),
          pl.BlockSpec((1, _H), lambda i: (0, 0)),
      ],
      out_specs=pl.BlockSpec((_BN, _H), lambda i: (i, 0)),
      out_shape=jax.ShapeDtypeStruct((_N, _H), jnp.float32),
  )(x, w, b.reshape(1, _H))


def _edge_body(a_ref, w0_ref, b0_ref, w1_ref, b1_ref, e0_ref, e1_ref):
  a = a_ref[...]
  e0_ref[...] = (
      jnp.dot(a, w0_ref[...], preferred_element_type=jnp.float32) + b0_ref[...]
  )
  e1_ref[...] = (
      jnp.dot(a, w1_ref[...], preferred_element_type=jnp.float32) + b1_ref[...]
  )


def _edge_embed(edge_attr, we0, be0, we1, be1):
  d_e = edge_attr.shape[1]
  return pl.pallas_call(
      _edge_body,
      grid=(_E // _BE,),
      in_specs=[
          pl.BlockSpec((_BE, d_e), lambda i: (i, 0)),
          pl.BlockSpec((d_e, _H), lambda i: (0, 0)),
          pl.BlockSpec((1, _H), lambda i: (0, 0)),
          pl.BlockSpec((d_e, _H), lambda i: (0, 0)),
          pl.BlockSpec((1, _H), lambda i: (0, 0)),
      ],
      out_specs=[
          pl.BlockSpec((_BE, _H), lambda i: (i, 0)),
          pl.BlockSpec((_BE, _H), lambda i: (i, 0)),
      ],
      out_shape=[
          jax.ShapeDtypeStruct((_E, _H), jnp.float32),
          jax.ShapeDtypeStruct((_E, _H), jnp.float32),
      ],
  )(edge_attr, we0, be0.reshape(1, _H), we1, be1.reshape(1, _H))


def _mlp_body(h_ref, agg_ref, scale_ref, w1_ref, b1_ref, w2_ref, b2_ref, o_ref):
  z = scale_ref[0, 0] * h_ref[...] + agg_ref[0] + agg_ref[1]
  t = jnp.maximum(
      jnp.dot(z, w1_ref[...], preferred_element_type=jnp.float32) + b1_ref[...],
      0.0,
  )
  o = jnp.dot(t, w2_ref[...], preferred_element_type=jnp.float32) + b2_ref[...]
  o_ref[...] = jnp.maximum(o, 0.0)


def _mlp_update(h, agg, eps, w1, b1, w2, b2):
  scale = (1.0 + eps).reshape(1, 1)
  h2 = 2 * _H
  return pl.pallas_call(
      _mlp_body,
      grid=(_N // _BN,),
      in_specs=[
          pl.BlockSpec((_BN, _H), lambda i: (i, 0)),
          pl.BlockSpec((_NC, _BN, _H), lambda i: (0, i, 0)),
          pl.BlockSpec((1, 1), lambda i: (0, 0)),
          pl.BlockSpec((_H, h2), lambda i: (0, 0)),
          pl.BlockSpec((1, h2), lambda i: (0, 0)),
          pl.BlockSpec((h2, _H), lambda i: (0, 0)),
          pl.BlockSpec((1, _H), lambda i: (0, 0)),
      ],
      out_specs=pl.BlockSpec((_BN, _H), lambda i: (i, 0)),
      out_shape=jax.ShapeDtypeStruct((_N, _H), jnp.float32),
  )(h, agg, scale, w1, b1.reshape(1, h2), w2, b2.reshape(1, _H))


def _final_body(h0_ref, h1_ref, h2_ref, w0_ref, w1_ref, w2_ref, b_ref, o_ref):
  o = jnp.dot(h0_ref[...], w0_ref[...], preferred_element_type=jnp.float32)
  o += jnp.dot(h1_ref[...], w1_ref[...], preferred_element_type=jnp.float32)
  o += jnp.dot(h2_ref[...], w2_ref[...], preferred_element_type=jnp.float32)
  o_ref[...] = o + b_ref[...]


def _final_proj(h0, h1, h2, w_out, b_out):
  return pl.pallas_call(
      _final_body,
      grid=(_N // _BN,),
      in_specs=[
          pl.BlockSpec((_BN, _H), lambda i: (i, 0)),
          pl.BlockSpec((_BN, _H), lambda i: (i, 0)),
          pl.BlockSpec((_BN, _H), lambda i: (i, 0)),
          pl.BlockSpec((_H, _H), lambda i: (0, 0)),
          pl.BlockSpec((_H, _H), lambda i: (0, 0)),
          pl.BlockSpec((_H, _H), lambda i: (0, 0)),
          pl.BlockSpec((1, _H), lambda i: (0, 0)),
      ],
      out_specs=pl.BlockSpec((_BN, _H), lambda i: (i, 0)),
      out_shape=jax.ShapeDtypeStruct((_N, _H), jnp.float32),
  )(h0, h1, h2, w_out[:_H], w_out[_H:2 * _H], w_out[2 * _H:], b_out.reshape(1, _H))


_mesh = plsc.VectorSubcoreMesh(core_axis_name="c", subcore_axis_name="s")


@functools.partial(
    pl.kernel,
    mesh=_mesh,
    out_type=jax.ShapeDtypeStruct((_NC * _NP, _H), jnp.float32),
    scratch_types=[
        pltpu.VMEM((_GE,), jnp.int32),
        pltpu.VMEM((_GCH, _K), jnp.int32),
        pltpu.VMEM((_K, _H), jnp.float32),
        pltpu.VMEM((_K, _H), jnp.float32),
        pltpu.VMEM((_ZR, _H), jnp.float32),
        pltpu.VMEM_SHARED((_NP, _H), jnp.float32),
        pltpu.SemaphoreType.DMA,
    ],
)
def _sc_agg(h_hbm, e_hbm, src_hbm, dst_hbm, out_hbm,
            src_v, dst_v, rows_v, e_v, z_v, acc_sh, sem):
  c = lax.axis_index("c")
  s = lax.axis_index("s")
  wid = s * _NC + c

  zero = jnp.zeros((16,), jnp.float32)

  def zfill(i, carry):
    r = i // 8
    q = (i % 8) * 16
    z_v[r, pl.ds(q, 16)] = zero
    return carry

  lax.fori_loop(0, _ZR * 8, zfill, 0)
  for t in range(_RPT // _ZR):
    pltpu.sync_copy(z_v, acc_sh.at[pl.ds(s * _RPT + t * _ZR, _ZR)])
  plsc.subcore_barrier()

  def group(g, gcarry):
    gid = wid * _NGRP + g
    pltpu.sync_copy(src_hbm.at[pl.ds(gid * _GE, _GE)], src_v)
    pltpu.sync_copy(dst_hbm.at[gid], dst_v)

    def chunk(j, carry):
      pltpu.async_copy(h_hbm.at[src_v.at[pl.ds(j * _K, _K)]], rows_v, sem).wait()
      pltpu.sync_copy(e_hbm.at[pl.ds((gid * _GCH + j) * _K, _K)], e_v)

      def row(r, rcarry):
        for q in range(0, _H, 16):
          hv = rows_v[r, pl.ds(q, 16)]
          ev = e_v[r, pl.ds(q, 16)]
          rows_v[r, pl.ds(q, 16)] = jnp.maximum(hv + ev, 0.0)
        return rcarry

      lax.fori_loop(0, _K, row, 0)
      pltpu.sync_copy(rows_v, acc_sh.at[dst_v.at[j]], add=True)
      return carry

    lax.fori_loop(0, _GCH, chunk, 0)
    return gcarry

  lax.fori_loop(0, _NGRP, group, 0)
  plsc.subcore_barrier()

  pltpu.sync_copy(
      acc_sh.at[pl.ds(s * _RPT, _RPT)],
      out_hbm.at[pl.ds(c * _NP + s * _RPT, _RPT)],
  )


def _sc_layer(h, e, src_r, dst_r):
  out = _sc_agg(h, e, src_r, dst_r)
  return out.reshape(_NC, _NP, _H)


def kernel(x, edge_index, edge_attr, w_node, b_node,
           we0, be0, w10, b10, w20, b20, eps0,
           we1, be1, w11, b11, w21, b21, eps1,
           w_out, b_out):
  src_r = edge_index[0]
  dst_r = edge_index[1].reshape(_NW * _NGRP, _GCH, _K)

  h0 = _node_encode(x, w_node, b_node)
  e0, e1 = _edge_embed(edge_attr, we0, be0, we1, be1)

  agg0 = _sc_layer(h0, e0, src_r, dst_r)
  h1 = _mlp_update(h0, agg0, eps0, w10, b10, w20, b20)

  agg1 = _sc_layer(h1, e1, src_r, dst_r)
  h2 = _mlp_update(h1, agg1, eps1, w11, b11, w21, b21)

  return _final_proj(h0, h1, h2, w_out, b_out)


# speedup vs baseline: 4.2867x; 1.5528x over previous
import functools

import jax
import jax.numpy as jnp
from jax import lax
from jax.experimental import pallas as pl
from jax.experimental.pallas import tpu as pltpu
from jax.experimental.pallas import tpu_sc as plsc

_N = 10000
_E = 320000
_H = 128
_NC = 2
_NS = 16
_NW = _NC * _NS
_EPW = _E // _NW
_K = 40
_NCH = _EPW // _K
_NP = 10240
_RPT = _NP // _NS
_NGRP = 5
_GCH = _NCH // _NGRP
_GE = _GCH * _K
_BN = 1000
_BE = 2000


def _node_enc_body(x_ref, w_ref, b_ref, o_ref):
  o_ref[...] = (
      jnp.dot(x_ref[...], w_ref[...], preferred_element_type=jnp.float32)
      + b_ref[...]
  )


def _node_encode(x, w, b):
  d_in = x.shape[1]
  return pl.pallas_call(
      _node_enc_body,
      grid=(_N // _BN,),
      in_specs=[
          pl.BlockSpec((_BN, d_in), lambda i: (i, 0)),
          pl.BlockSpec((d_in, _H), lambda i: (0, 0)),
          pl.BlockSpec((1, _H), lambda i: (0, 0)),
      ],
      out_specs=pl.BlockSpec((_BN, _H), lambda i: (i, 0)),
      out_shape=jax.ShapeDtypeStruct((_N, _H), jnp.float32),
  )(x, w, b.reshape(1, _H))


def _edge_body(a_ref, w_ref, b_ref, e_ref):
  e_ref[...] = (
      jnp.dot(a_ref[...], w_ref[...], preferred_element_type=jnp.float32)
      + b_ref[...]
  )


def _edge_embed(edge_attr, we, be):
  d_e = edge_attr.shape[1]
  return pl.pallas_call(
      _edge_body,
      grid=(_E // _BE,),
      in_specs=[
          pl.BlockSpec((_BE, d_e), lambda i: (i, 0)),
          pl.BlockSpec((d_e, _H), lambda i: (0, 0)),
          pl.BlockSpec((1, _H), lambda i: (0, 0)),
      ],
      out_specs=pl.BlockSpec((_BE, _H), lambda i: (i, 0)),
      out_shape=jax.ShapeDtypeStruct((_E, _H), jnp.float32),
  )(edge_attr, we, be.reshape(1, _H))


def _mlp_body(h_ref, agg_ref, scale_ref, w1_ref, b1_ref, w2_ref, b2_ref, o_ref):
  z = scale_ref[0, 0] * h_ref[...] + agg_ref[0] + agg_ref[1]
  t = jnp.maximum(
      jnp.dot(z, w1_ref[...], preferred_element_type=jnp.float32) + b1_ref[...],
      0.0,
  )
  o = jnp.dot(t, w2_ref[...], preferred_element_type=jnp.float32) + b2_ref[...]
  o_ref[...] = jnp.maximum(o, 0.0)


def _mlp_update(h, agg, eps, w1, b1, w2, b2):
  scale = (1.0 + eps).reshape(1, 1)
  h2 = 2 * _H
  return pl.pallas_call(
      _mlp_body,
      grid=(_N // _BN,),
      in_specs=[
          pl.BlockSpec((_BN, _H), lambda i: (i, 0)),
          pl.BlockSpec((_NC, _BN, _H), lambda i: (0, i, 0)),
          pl.BlockSpec((1, 1), lambda i: (0, 0)),
          pl.BlockSpec((_H, h2), lambda i: (0, 0)),
          pl.BlockSpec((1, h2), lambda i: (0, 0)),
          pl.BlockSpec((h2, _H), lambda i: (0, 0)),
          pl.BlockSpec((1, _H), lambda i: (0, 0)),
      ],
      out_specs=pl.BlockSpec((_BN, _H), lambda i: (i, 0)),
      out_shape=jax.ShapeDtypeStruct((_N, _H), jnp.float32),
  )(h, agg, scale, w1, b1.reshape(1, h2), w2, b2.reshape(1, _H))


def _final_body(h0_ref, h1_ref, h2_ref, w0_ref, w1_ref, w2_ref, b_ref, o_ref):
  o = jnp.dot(h0_ref[...], w0_ref[...], preferred_element_type=jnp.float32)
  o += jnp.dot(h1_ref[...], w1_ref[...], preferred_element_type=jnp.float32)
  o += jnp.dot(h2_ref[...], w2_ref[...], preferred_element_type=jnp.float32)
  o_ref[...] = o + b_ref[...]


def _final_proj(h0, h1, h2, w_out, b_out):
  return pl.pallas_call(
      _final_body,
      grid=(_N // _BN,),
      in_specs=[
          pl.BlockSpec((_BN, _H), lambda i: (i, 0)),
          pl.BlockSpec((_BN, _H), lambda i: (i, 0)),
          pl.BlockSpec((_BN, _H), lambda i: (i, 0)),
          pl.BlockSpec((_H, _H), lambda i: (0, 0)),
          pl.BlockSpec((_H, _H), lambda i: (0, 0)),
          pl.BlockSpec((_H, _H), lambda i: (0, 0)),
          pl.BlockSpec((1, _H), lambda i: (0, 0)),
      ],
      out_specs=pl.BlockSpec((_BN, _H), lambda i: (i, 0)),
      out_shape=jax.ShapeDtypeStruct((_N, _H), jnp.float32),
  )(h0, h1, h2, w_out[:_H], w_out[_H:2 * _H], w_out[2 * _H:], b_out.reshape(1, _H))


_mesh = plsc.VectorSubcoreMesh(core_axis_name="c", subcore_axis_name="s")


@functools.partial(
    pl.kernel,
    mesh=_mesh,
    out_type=jax.ShapeDtypeStruct((_NC * _NP, _H), jnp.float32),
    scratch_types=[
        pltpu.VMEM((_GE,), jnp.int32),
        pltpu.VMEM((_GCH, _K), jnp.int32),
        pltpu.VMEM((2, _K, _H), jnp.float32),
        pltpu.VMEM((2, _K, _H), jnp.float32),
        pltpu.VMEM((2, _K, _H), jnp.float32),
        pltpu.VMEM_SHARED((_NP, _H), jnp.float32),
        pltpu.SemaphoreType.DMA((2,)),
        pltpu.SemaphoreType.DMA((2,)),
        pltpu.SemaphoreType.DMA((2,)),
    ],
)
def _sc_agg(h_hbm, e_hbm, src_hbm, dst_hbm, out_hbm,
            src_v, dst_v, rows_v, e_v, s_v, acc_sh, gsem, esem, ssem):
  c = lax.axis_index("c")
  s = lax.axis_index("s")
  wid = s * _NC + c

  zero = jnp.zeros((16,), jnp.float32)

  def zfill(i, carry):
    r = i // 8
    q = (i % 8) * 16
    s_v[0, r, pl.ds(q, 16)] = zero
    return carry

  lax.fori_loop(0, _K * 8, zfill, 0)
  for t in range(_RPT // _K):
    pltpu.sync_copy(s_v.at[0], acc_sh.at[pl.ds(s * _RPT + t * _K, _K)])
  plsc.subcore_barrier()

  def issue_in(gid, j, b):
    pltpu.async_copy(
        h_hbm.at[src_v.at[pl.ds(j * _K, _K)]], rows_v.at[b], gsem.at[b])
    pltpu.async_copy(
        e_hbm.at[pl.ds((gid * _GCH + j) * _K, _K)], e_v.at[b], esem.at[b])

  def wait_in(gid, j, b):
    pltpu.make_async_copy(
        h_hbm.at[src_v.at[pl.ds(j * _K, _K)]], rows_v.at[b], gsem.at[b]).wait()
    pltpu.make_async_copy(
        e_hbm.at[pl.ds((gid * _GCH + j) * _K, _K)], e_v.at[b], esem.at[b]).wait()

  def issue_sc(j, b):
    pltpu.async_copy(s_v.at[b], acc_sh.at[dst_v.at[j]], ssem.at[b], add=True)

  def wait_sc(j, b):
    pltpu.make_async_copy(s_v.at[b], acc_sh.at[dst_v.at[j]], ssem.at[b]).wait()

  def compute(b):
    def row(r, rcarry):
      for q in range(0, _H, 16):
        hv = rows_v[b, r, pl.ds(q, 16)]
        ev = e_v[b, r, pl.ds(q, 16)]
        s_v[b, r, pl.ds(q, 16)] = jnp.maximum(hv + ev, 0.0)
      return rcarry

    lax.fori_loop(0, _K, row, 0)

  def group(g, gcarry):
    gid = wid * _NGRP + g
    pltpu.sync_copy(src_hbm.at[pl.ds(gid * _GE, _GE)], src_v)
    pltpu.sync_copy(dst_hbm.at[gid], dst_v)

    for b in range(2):
      issue_in(gid, b, b)
    for b in range(2):
      wait_in(gid, b, b)
      compute(b)
      issue_in(gid, b + 2, b)
      issue_sc(b, b)

    def pair(j2, carry):
      for b in range(2):
        j = 2 * j2 + b
        wait_sc(j - 2, b)
        wait_in(gid, j, b)
        compute(b)
        issue_in(gid, j + 2, b)
        issue_sc(j, b)
      return carry

    lax.fori_loop(1, _GCH // 2 - 1, pair, 0)

    for b in range(2):
      j = _GCH - 2 + b
      wait_sc(j - 2, b)
      wait_in(gid, j, b)
      compute(b)
      issue_sc(j, b)
    for b in range(2):
      wait_sc(_GCH - 2 + b, b)
    return gcarry

  lax.fori_loop(0, _NGRP, group, 0)
  plsc.subcore_barrier()

  pltpu.sync_copy(
      acc_sh.at[pl.ds(s * _RPT, _RPT)],
      out_hbm.at[pl.ds(c * _NP + s * _RPT, _RPT)],
  )


def _sc_layer(h, e, src_r, dst_r):
  out = _sc_agg(h, e, src_r, dst_r)
  return out.reshape(_NC, _NP, _H)


def kernel(x, edge_index, edge_attr, w_node, b_node,
           we0, be0, w10, b10, w20, b20, eps0,
           we1, be1, w11, b11, w21, b21, eps1,
           w_out, b_out):
  src_r = edge_index[0]
  dst_r = edge_index[1].reshape(_NW * _NGRP, _GCH, _K)

  h0 = _node_encode(x, w_node, b_node)
  e0 = _edge_embed(edge_attr, we0, be0)
  e1 = _edge_embed(edge_attr, we1, be1)

  agg0 = _sc_layer(h0, e0, src_r, dst_r)
  h1 = _mlp_update(h0, agg0, eps0, w10, b10, w20, b20)

  agg1 = _sc_layer(h1, e1, src_r, dst_r)
  h2 = _mlp_update(h1, agg1, eps1, w11, b11, w21, b21)

  return _final_proj(h0, h1, h2, w_out, b_out)
